# baseline (speedup 1.0000x reference)
import jax
import jax.numpy as jnp
from jax import lax
from jax.experimental import pallas as pl
from jax.experimental.pallas import tpu as pltpu
from jax.experimental.pallas import tpu_sc as plsc

N_ROWS = 160000
N_COLS = 256
N_SEG = 10000
S_PAD = 10240
NC = 2
NS = 16
L = 16
DC = N_COLS // NC
RPT = N_ROWS // NS
CH = 80
NCH = RPT // CH
NBUF = 4
SEG_PT = S_PAD // NS
OB = 80
NOB = SEG_PT // OB


def _scatter_mean_body(x_hbm, idx_hbm, out_hbm,
                       acc_sh, cnt_sh, ones_buf, cbuf,
                       x_bufs, idx_bufs, x_sems, i_sems, s_sems):
    c = lax.axis_index("c")
    s = lax.axis_index("s")
    col0 = c * DC
    row0 = s * RPT
    seg0 = s * SEG_PT

    zv = jnp.zeros((L,), jnp.float32)
    onev = jnp.ones((L,), jnp.float32)

    def start_load(k, b):
        r0 = pl.multiple_of(row0 + k * CH, 8)
        pltpu.async_copy(idx_hbm.at[pl.ds(r0, CH)], idx_bufs[b], i_sems[b])
        pltpu.async_copy(x_hbm.at[pl.ds(r0, CH), pl.ds(col0, DC)],
                         x_bufs[b], x_sems[b])

    for b in range(1, NBUF):
        start_load(b, b)

    def fill_ones(i, carry):
        ones_buf[pl.ds(i * L, L)] = onev
        return carry
    lax.fori_loop(0, CH // L, fill_ones, 0)

    obuf = x_bufs[0]

    def zero_blk(i, carry):
        for jj in range(DC // L):
            obuf[i, pl.ds(jj * L, L)] = zv
        return carry
    lax.fori_loop(0, OB, zero_blk, 0)

    def zero_cnt(i, carry):
        cbuf[pl.ds(i * L, L)] = zv
        return carry
    lax.fori_loop(0, OB // L, zero_cnt, 0)

    for m in range(NOB):
        pltpu.sync_copy(obuf, acc_sh.at[pl.ds(seg0 + m * OB, OB), :])
        pltpu.sync_copy(cbuf, cnt_sh.at[pl.ds(seg0 + m * OB, OB)])

    plsc.subcore_barrier()

    start_load(0, 0)

    def wait_load(b):
        pltpu.make_async_copy(idx_hbm.at[pl.ds(0, CH)],
                              idx_bufs[b], i_sems[b]).wait()
        pltpu.make_async_copy(
            x_hbm.at[pl.ds(0, CH), pl.ds(col0, DC)],
            x_bufs[b], x_sems[b]).wait()

    def scatter(k, b):
        d = pltpu.async_copy(ones_buf, cnt_sh.at[idx_bufs[b]],
                             x_sems[NBUF], add=True)
        pltpu.sync_copy(x_bufs[b], acc_sh.at[idx_bufs[b]], add=True)
        d.wait()

    NTRI = NCH // NBUF

    def tri(i, carry):
        ds = []
        for j in range(NBUF):
            wait_load(j)
            ds.append((
                pltpu.async_copy(ones_buf, cnt_sh.at[idx_bufs[j]],
                                 s_sems[j], add=True),
                pltpu.async_copy(x_bufs[j], acc_sh.at[idx_bufs[j]],
                                 s_sems[j], add=True)))
        for j in range(NBUF):
            k = NBUF * i + j
            d1, d2 = ds[j]
            d1.wait()
            d2.wait()

            @pl.when(k + NBUF < NCH)
            def _():
                start_load(k + NBUF, j)
        return carry
    lax.fori_loop(0, NTRI, tri, 0)
    for j in range(NTRI * NBUF, NCH):
        b = j % NBUF
        wait_load(b)
        scatter(j, b)
    plsc.subcore_barrier()

    def finalize(m, carry):
        g0 = seg0 + m * OB
        pltpu.sync_copy(acc_sh.at[pl.ds(g0, OB), :], obuf)
        pltpu.sync_copy(cnt_sh.at[pl.ds(g0, OB)], cbuf)

        def rowfix(g, inner):
            cv = jnp.maximum(cbuf[pl.ds(g * L, L)], 1.0)
            rv = jnp.full((L,), 1.0, jnp.float32) / cv
            for j in range(L):
                rvec = jnp.full((L,), rv[j], jnp.float32)
                row = g * L + j
                for jj in range(DC // L):
                    sl = pl.ds(jj * L, L)
                    obuf[row, sl] = obuf[row, sl] * rvec
            return inner
        lax.fori_loop(0, OB // L, rowfix, 0)
        pltpu.sync_copy(obuf, out_hbm.at[pl.ds(g0, OB), pl.ds(col0, DC)])
        return carry
    lax.fori_loop(0, NOB, finalize, 0)


@jax.jit
def kernel(x, index):
    idx32 = index.astype(jnp.int32)
    mesh = plsc.VectorSubcoreMesh(core_axis_name="c", subcore_axis_name="s")
    f = pl.kernel(
        _scatter_mean_body,
        out_type=jax.ShapeDtypeStruct((S_PAD, N_COLS), jnp.float32),
        mesh=mesh,
        scratch_types=[
            pltpu.VMEM_SHARED((S_PAD, DC), jnp.float32),
            pltpu.VMEM_SHARED((S_PAD,), jnp.float32),
            pltpu.VMEM((CH,), jnp.float32),
            pltpu.VMEM((OB,), jnp.float32),
            [pltpu.VMEM((CH, DC), jnp.float32)] * NBUF,
            [pltpu.VMEM((CH,), jnp.int32)] * NBUF,
            [pltpu.SemaphoreType.DMA] * (NBUF + 1),
            [pltpu.SemaphoreType.DMA] * NBUF,
            [pltpu.SemaphoreType.DMA] * NBUF,
        ],
    )
    return f(x, idx32)[:N_SEG]
